# Initial kernel scaffold; baseline (speedup 1.0000x reference)
import jax
import jax.numpy as jnp
from jax.experimental import pallas as pl


def kernel(atom_inputs, tables, ringsize_table, aroma_table, fused_table):
    raise NotImplementedError("write your pallas kernel here")


# speedup vs baseline: 7.0760x; 7.0760x over previous
import functools

import jax
import jax.numpy as jnp
from jax import lax
from jax.experimental import pallas as pl
from jax.experimental.pallas import tpu as pltpu
from jax.experimental.pallas import tpu_sc as plsc

NUM_PLAIN = 27
PLAIN_VOCAB = 64
EMBED_DIM = 16
RING_VALS = [0, 3, 4, 5, 6, 7, 8]

N_ROWS = 100000
N_FIELDS = 30
TOTAL = N_ROWS * N_FIELDS

NC, NS, L = 2, 16, 16
NW = NC * NS

K = 128
PB = 4
BLK = K * PB
BPW = 184
PER_W = BPW * BLK
PADDED = NW * PER_W
NBLK128 = PADDED // K

RING_BASE = NUM_PLAIN * PLAIN_VOCAB
AROMA_BASE = RING_BASE + len(RING_VALS)
FUSED_BASE = AROMA_BASE + 2


def _row_index(v, pos):
    f = pos % N_FIELDS
    c63 = jnp.minimum(jnp.maximum(v, 0), PLAIN_VOCAB - 1)
    c1 = jnp.minimum(jnp.maximum(v, 0), 1)
    plain = f * PLAIN_VOCAB + c63
    m = jnp.zeros_like(v)
    for i, rv in enumerate(RING_VALS):
        if rv == 0:
            continue
        m = jnp.where(v == rv, i, m)
    ring = RING_BASE + m
    aroma = AROMA_BASE + c1
    fused = FUSED_BASE + c1
    out = jnp.where(f == NUM_PLAIN, ring, plain)
    out = jnp.where(f == NUM_PLAIN + 1, aroma, out)
    out = jnp.where(f == NUM_PLAIN + 2, fused, out)
    return out


def _sc_body(tab_hbm, in_hbm, out_hbm, vals_v, idx_v, rows_v, sem):
    wid = lax.axis_index("s") * NC + lax.axis_index("c")
    base128 = wid * (BPW * PB)
    lane = lax.iota(jnp.int32, L)

    def body(g, carry):
        blk = base128 + g * PB
        pltpu.sync_copy(in_hbm.at[pl.ds(blk, PB)], vals_v)
        for q in range(PB):
            for r in range(K // L):
                v = vals_v[q, pl.ds(r * L, L)]
                pos = (blk + q) * K + r * L + lane
                idx_v[q, pl.ds(r * L, L)] = _row_index(v, pos)
        copies = [
            pltpu.async_copy(tab_hbm.at[idx_v.at[q]], rows_v.at[q], sem)
            for q in range(PB)
        ]
        for c in copies:
            c.wait()
        pltpu.sync_copy(rows_v, out_hbm.at[pl.ds(blk, PB)])
        return carry

    lax.fori_loop(0, BPW, body, 0)


@functools.partial(jax.jit, static_argnames=())
def _sc_gather(flat_tab, flat_in):
    mesh = plsc.VectorSubcoreMesh(core_axis_name="c", subcore_axis_name="s")
    fn = pl.kernel(
        _sc_body,
        mesh=mesh,
        compiler_params=pltpu.CompilerParams(use_tc_tiling_on_sc=False),
        out_type=jax.ShapeDtypeStruct((NBLK128, K, EMBED_DIM), jnp.float32),
        scratch_types=[
            pltpu.VMEM((PB, K), jnp.int32),
            pltpu.VMEM((PB, K), jnp.int32),
            pltpu.VMEM((PB, K, EMBED_DIM), jnp.float32),
            pltpu.SemaphoreType.DMA,
        ],
    )
    return fn(flat_tab, flat_in)


def kernel(atom_inputs, tables, ringsize_table, aroma_table, fused_table):
    n, nf = atom_inputs.shape
    assert n * nf == TOTAL
    flat_tab = jnp.concatenate(
        [tables.reshape(-1, EMBED_DIM), ringsize_table, aroma_table, fused_table],
        axis=0,
    )
    flat_in = jnp.pad(atom_inputs.reshape(-1), (0, PADDED - TOTAL))
    out = _sc_gather(flat_tab, flat_in.reshape(NBLK128, K))
    return out.reshape(-1, EMBED_DIM)[:TOTAL].reshape(n, nf * EMBED_DIM)


# speedup vs baseline: 8.5037x; 1.2018x over previous
import functools

import jax
import jax.numpy as jnp
from jax import lax
from jax.experimental import pallas as pl
from jax.experimental.pallas import tpu as pltpu
from jax.experimental.pallas import tpu_sc as plsc

NUM_PLAIN = 27
PLAIN_VOCAB = 64
EMBED_DIM = 16
RING_VALS = [0, 3, 4, 5, 6, 7, 8]

N_ROWS = 100000
N_FIELDS = 30
TOTAL = N_ROWS * N_FIELDS

NC, NS, L = 2, 16, 16
NW = NC * NS

K = 128
PB = 4
BLK = K * PB
BPW = 184
PER_W = BPW * BLK
PADDED = NW * PER_W
NBLK128 = PADDED // K

RING_BASE = NUM_PLAIN * PLAIN_VOCAB
AROMA_BASE = RING_BASE + len(RING_VALS)
FUSED_BASE = AROMA_BASE + 2


def _row_index(v, pos):
    f = pos % N_FIELDS
    c63 = jnp.minimum(jnp.maximum(v, 0), PLAIN_VOCAB - 1)
    c1 = jnp.minimum(jnp.maximum(v, 0), 1)
    plain = f * PLAIN_VOCAB + c63
    m = jnp.zeros_like(v)
    for i, rv in enumerate(RING_VALS):
        if rv == 0:
            continue
        m = jnp.where(v == rv, i, m)
    ring = RING_BASE + m
    aroma = AROMA_BASE + c1
    fused = FUSED_BASE + c1
    out = jnp.where(f == NUM_PLAIN, ring, plain)
    out = jnp.where(f == NUM_PLAIN + 1, aroma, out)
    out = jnp.where(f == NUM_PLAIN + 2, fused, out)
    return out


def _sc_body(tab_hbm, in_hbm, out_hbm, tab_v, vals_v, idx_v, rows_v, sem):
    wid = lax.axis_index("s") * NC + lax.axis_index("c")
    base128 = wid * (BPW * PB)
    lane = lax.iota(jnp.int32, L)

    @pl.when(lax.axis_index("s") == 0)
    def _stage():
        pltpu.sync_copy(tab_hbm, tab_v)

    plsc.subcore_barrier()

    def body(g, carry):
        blk = base128 + g * PB
        pltpu.sync_copy(in_hbm.at[pl.ds(blk, PB)], vals_v)
        for q in range(PB):
            for r in range(K // L):
                v = vals_v[q, pl.ds(r * L, L)]
                pos = (blk + q) * K + r * L + lane
                idx_v[q, pl.ds(r * L, L)] = _row_index(v, pos)
        copies = [
            pltpu.async_copy(tab_v.at[idx_v.at[q]], rows_v.at[q], sem)
            for q in range(PB)
        ]
        for c in copies:
            c.wait()
        pltpu.sync_copy(rows_v, out_hbm.at[pl.ds(blk, PB)])
        return carry

    lax.fori_loop(0, BPW, body, 0)


@functools.partial(jax.jit, static_argnames=())
def _sc_gather(flat_tab, flat_in):
    mesh = plsc.VectorSubcoreMesh(core_axis_name="c", subcore_axis_name="s")
    fn = pl.kernel(
        _sc_body,
        mesh=mesh,
        compiler_params=pltpu.CompilerParams(use_tc_tiling_on_sc=False),
        out_type=jax.ShapeDtypeStruct((NBLK128, K, EMBED_DIM), jnp.float32),
        scratch_types=[
            pltpu.VMEM_SHARED((1739, EMBED_DIM), jnp.float32),
            pltpu.VMEM((PB, K), jnp.int32),
            pltpu.VMEM((PB, K), jnp.int32),
            pltpu.VMEM((PB, K, EMBED_DIM), jnp.float32),
            pltpu.SemaphoreType.DMA,
        ],
    )
    return fn(flat_tab, flat_in)


def kernel(atom_inputs, tables, ringsize_table, aroma_table, fused_table):
    n, nf = atom_inputs.shape
    assert n * nf == TOTAL
    flat_tab = jnp.concatenate(
        [tables.reshape(-1, EMBED_DIM), ringsize_table, aroma_table, fused_table],
        axis=0,
    )
    flat_in = jnp.pad(atom_inputs.reshape(-1), (0, PADDED - TOTAL))
    out = _sc_gather(flat_tab, flat_in.reshape(NBLK128, K))
    return out.reshape(-1, EMBED_DIM)[:TOTAL].reshape(n, nf * EMBED_DIM)
